# baseline (speedup 1.0000x reference)
import jax
import jax.numpy as jnp
import numpy as np
from jax import lax
from jax.experimental import pallas as pl
from jax.experimental.pallas import tpu as pltpu
from jax.experimental.pallas import tpu_sc as plsc

L = 16
T = 1048576
F = 2
WIDTH = 64
OUT = 3
N = 524288
PRIME = 2654435761
PRIME_I32 = np.int32(np.uint32(PRIME).astype(np.int64) - (1 << 32))
MASK = T - 1
HI_I32 = np.int32(np.int64(0xFFFF0000) - (1 << 32))

_b = np.exp((np.log(2048) - np.log(16)) / (L - 1))
_RES = np.floor(16 * _b ** np.arange(L)).astype(np.int64)

NC, NS, LANES = 2, 16, 16
NW = NC * NS
C = 256
NPW = N // NW
NCHUNK = NPW // C
IDXB = 4 * C
GRPB = IDXB * 8

NCACHE = 8
CELLW = [int(r) + 1 for r in _RES[:NCACHE]]
CELLS = [w * w for w in CELLW]
CELLS16 = [-(-c // 16) * 16 for c in CELLS]
GBASE = [0] * NCACHE
for _i in range(1, NCACHE):
    GBASE[_i] = GBASE[_i - 1] + CELLS16[_i - 1]
GRID_WORDS = GBASE[-1] + CELLS16[-1] + 128


def _encode_body(xT_hbm, resb_hbm, tabu_hbm, out_hbm, xv, idxv, b8v, grpv,
                 featv, resv, gridv, sem):
    wid = lax.axis_index("s") * NC + lax.axis_index("c")
    base0 = wid * NPW
    pltpu.sync_copy(resb_hbm, resv)
    iota = lax.iota(jnp.int32, LANES)

    def unpack0(u):
        return plsc.bitcast(u << 16, jnp.float32)

    def unpack1(u):
        return plsc.bitcast(u & HI_I32, jnp.float32)

    for lc in range(NCACHE):
        W = CELLW[lc]
        ncell = CELLS[lc]
        gb = GBASE[lc]
        nsg = -(-ncell // 128)

        @pl.loop(0, nsg)
        def _sg(sg, _lc=lc, _W=W, _nc=ncell, _gb=gb):
            cbase = sg * 128
            for g in range(8):
                cell = jnp.minimum(cbase + g * LANES + iota, _nc - 1)
                gy = cell // _W
                gx = cell - gy * _W
                h = ((gx ^ (gy * PRIME_I32)) & MASK) + _lc * T
                idxv[pl.ds(g * LANES, LANES)] = h

            @pl.loop(0, 8)
            def _q(q):
                hv = idxv[pl.ds(LANES * q, LANES)]
                for t in range(LANES):
                    b8 = (hv[t] // 4) * 8
                    pltpu.async_copy(
                        tabu_hbm.at[pl.ds(b8, 8)],
                        grpv.at[pl.ds((LANES * q + t) * 8, 8)], sem)

            pltpu.make_async_copy(tabu_hbm.at[pl.ds(0, 1024)],
                                  grpv.at[pl.ds(0, 1024)], sem).wait()
            for g in range(8):
                hv = idxv[pl.ds(g * LANES, LANES)]
                e = (g * LANES + iota) * 8 + ((hv * 2) & 7)
                q0 = plsc.bitcast(plsc.load_gather(grpv, [e]), jnp.int32)
                q1 = plsc.bitcast(plsc.load_gather(grpv, [e + 1]), jnp.int32)
                gridv[pl.ds(_gb + cbase + g * LANES, LANES)] = \
                    ((q0 >> 16) & 0xFFFF) | (q1 & HI_I32)

    def compute_idx(l, lb):
        rv = resv[pl.ds(l * LANES, LANES)]
        off = l * T

        @pl.loop(0, C // LANES)
        def _grp(g):
            s = g * LANES
            px = xv[pl.ds(s, LANES)] * rv
            py = xv[pl.ds(C + s, LANES)] * rv
            ix = px.astype(jnp.int32)
            iy = py.astype(jnp.int32)
            m0 = iy * PRIME_I32
            m1 = m0 + PRIME_I32
            x1 = ix + 1
            b = lb * IDXB + s
            h0 = ((ix ^ m0) & MASK) + off
            h1 = ((x1 ^ m0) & MASK) + off
            h2 = ((ix ^ m1) & MASK) + off
            h3 = ((x1 ^ m1) & MASK) + off
            idxv[pl.ds(b + 0 * C, LANES)] = h0
            idxv[pl.ds(b + 1 * C, LANES)] = h1
            idxv[pl.ds(b + 2 * C, LANES)] = h2
            idxv[pl.ds(b + 3 * C, LANES)] = h3
            m4 = jnp.int32(~3)
            bb = lb * IDXB + s
            b8v[pl.ds(bb + 0 * C, LANES)] = (h0 & m4) * 2
            b8v[pl.ds(bb + 1 * C, LANES)] = (h1 & m4) * 2
            b8v[pl.ds(bb + 2 * C, LANES)] = (h2 & m4) * 2
            b8v[pl.ds(bb + 3 * C, LANES)] = (h3 & m4) * 2

    def fire(lb):
        @pl.loop(0, IDXB // LANES, unroll=4)
        def _q(q):
            bv = b8v[pl.ds(lb * IDXB + LANES * q, LANES)]
            gbase = lb * GRPB + LANES * q * 8
            for t in range(LANES):
                pltpu.async_copy(
                    tabu_hbm.at[pl.ds(pl.multiple_of(bv[t], 8), 8)],
                    grpv.at[pl.ds(gbase + t * 8, 8)], sem)

    def drain():
        pltpu.make_async_copy(tabu_hbm.at[pl.ds(0, GRPB)],
                              grpv.at[pl.ds(0, GRPB)], sem).wait()

    def interp(lp, pb):
        rv = resv[pl.ds(lp * LANES, LANES)]

        @pl.loop(0, C // LANES)
        def _grp(g):
            s = g * LANES
            px = xv[pl.ds(s, LANES)] * rv
            py = xv[pl.ds(C + s, LANES)] * rv
            fx = px - px.astype(jnp.int32).astype(jnp.float32)
            fy = py - py.astype(jnp.int32).astype(jnp.float32)
            wx0 = 1.0 - fx
            wy0 = 1.0 - fy
            ws = (wx0 * wy0, fx * wy0, wx0 * fy, fx * fy)
            acc0 = jnp.zeros((LANES,), jnp.float32)
            acc1 = jnp.zeros((LANES,), jnp.float32)
            for c in range(4):
                hv = idxv[pl.ds(pb * IDXB + c * C + s, LANES)]
                eidx = (pb * GRPB + (c * C + s) * 8 + iota * 8
                        + ((hv * 2) & 7))
                f0 = plsc.load_gather(grpv, [eidx])
                f1 = plsc.load_gather(grpv, [eidx + 1])
                acc0 = acc0 + f0 * ws[c]
                acc1 = acc1 + f1 * ws[c]
            fidx = (s + iota) * (2 * L) + 2 * lp
            plsc.store_scatter(featv, [fidx], acc0)
            plsc.store_scatter(featv, [fidx + 1], acc1)

    def interp_cached(lc):
        rv = resv[pl.ds(lc * LANES, LANES)]
        W = CELLW[lc]
        gb = GBASE[lc]

        @pl.loop(0, C // LANES)
        def _grp(g):
            s = g * LANES
            px = xv[pl.ds(s, LANES)] * rv
            py = xv[pl.ds(C + s, LANES)] * rv
            ix = px.astype(jnp.int32)
            iy = py.astype(jnp.int32)
            fx = px - ix.astype(jnp.float32)
            fy = py - iy.astype(jnp.float32)
            wx0 = 1.0 - fx
            wy0 = 1.0 - fy
            c00 = iy * W + ix + gb
            u00 = plsc.load_gather(gridv, [c00])
            u10 = plsc.load_gather(gridv, [c00 + 1])
            u01 = plsc.load_gather(gridv, [c00 + W])
            u11 = plsc.load_gather(gridv, [c00 + W + 1])
            w00 = wx0 * wy0
            w10 = fx * wy0
            w01 = wx0 * fy
            w11 = fx * fy
            acc0 = (unpack0(u00) * w00 + unpack0(u10) * w10
                    + unpack0(u01) * w01 + unpack0(u11) * w11)
            acc1 = (unpack1(u00) * w00 + unpack1(u10) * w10
                    + unpack1(u01) * w01 + unpack1(u11) * w11)
            fidx = (s + iota) * (2 * L) + 2 * lc
            plsc.store_scatter(featv, [fidx], acc0)
            plsc.store_scatter(featv, [fidx + 1], acc1)

    @pl.loop(0, NCHUNK)
    def _chunk(ci):
        base = base0 + ci * C
        pltpu.sync_copy(xT_hbm.at[0, pl.ds(base, C)], xv.at[pl.ds(0, C)])
        pltpu.sync_copy(xT_hbm.at[1, pl.ds(base, C)], xv.at[pl.ds(C, C)])

        compute_idx(NCACHE, NCACHE & 1)
        fire(NCACHE & 1)

        for lc in range(NCACHE):
            interp_cached(lc)

        @pl.loop(NCACHE + 1, L)
        def _lvl(l):
            lb = l & 1
            compute_idx(l, lb)
            fire(lb)
            drain()
            interp(l - 1, 1 - lb)

        drain()
        interp(L - 1, (L - 1) & 1)

        pltpu.sync_copy(featv, out_hbm.at[pl.ds(base * 2 * L, C * 2 * L)])


def _encode_sc(xT, tabu):
    mesh = plsc.VectorSubcoreMesh(core_axis_name="c", subcore_axis_name="s",
                                  num_cores=NC, num_subcores=NS)
    resb = jnp.asarray(
        np.broadcast_to(_RES.astype(np.float32)[:, None],
                        (L, LANES)).reshape(-1))
    enc = pl.kernel(
        _encode_body,
        out_type=jax.ShapeDtypeStruct((N * 2 * L,), jnp.float32),
        mesh=mesh,
        scratch_types=[
            pltpu.VMEM((2 * C,), jnp.float32),
            pltpu.VMEM((2 * IDXB,), jnp.int32),
            pltpu.VMEM((2 * IDXB,), jnp.int32),
            pltpu.VMEM((2 * GRPB,), jnp.float32),
            pltpu.VMEM((C * 2 * L,), jnp.float32),
            pltpu.VMEM((L * LANES,), jnp.float32),
            pltpu.VMEM((GRID_WORDS,), jnp.int32),
            pltpu.SemaphoreType.DMA,
        ],
        compiler_params=pltpu.CompilerParams(needs_layout_passes=False),
    )
    return enc(xT, resb, tabu).reshape(N, 2 * L)


def _mlp_body(f_ref, w1_ref, b1_ref, w2_ref, b2_ref, w3_ref, b3_ref, o_ref):
    h = f_ref[...]
    h = jnp.maximum(
        jnp.dot(h, w1_ref[...], preferred_element_type=jnp.float32)
        + b1_ref[...], 0.0)
    h = jnp.maximum(
        jnp.dot(h, w2_ref[...], preferred_element_type=jnp.float32)
        + b2_ref[...], 0.0)
    y = jnp.dot(h, w3_ref[...], preferred_element_type=jnp.float32) \
        + b3_ref[...]
    o_ref[...] = jax.nn.sigmoid(y)


def _mlp_tc(feats, Ws, bs):
    NB = 4096
    grid = (N // NB,)
    full = lambda shape: pl.BlockSpec(shape, lambda i: (0, 0))
    return pl.pallas_call(
        _mlp_body,
        grid=grid,
        in_specs=[
            pl.BlockSpec((NB, 2 * L), lambda i: (i, 0)),
            full((2 * L, WIDTH)),
            full((1, WIDTH)),
            full((WIDTH, WIDTH)),
            full((1, WIDTH)),
            full((WIDTH, OUT)),
            full((1, OUT)),
        ],
        out_specs=pl.BlockSpec((NB, OUT), lambda i: (i, 0)),
        out_shape=jax.ShapeDtypeStruct((N, OUT), jnp.float32),
    )(feats, Ws[0], bs[0].reshape(1, -1), Ws[1], bs[1].reshape(1, -1),
      Ws[2], bs[2].reshape(1, -1))


def kernel(x, table, Ws, bs):
    xT = x.T
    tabf = table.reshape(-1)
    feats = _encode_sc(xT, tabf)
    return _mlp_tc(feats, Ws, bs)


# speedup vs baseline: 7.7917x; 7.7917x over previous
import jax
import jax.numpy as jnp
import numpy as np
from jax import lax
from jax.experimental import pallas as pl
from jax.experimental.pallas import tpu as pltpu
from jax.experimental.pallas import tpu_sc as plsc

L = 16
T = 1048576
F = 2
WIDTH = 64
OUT = 3
N = 524288
PRIME = 2654435761
PRIME_I32 = np.int32(np.uint32(PRIME).astype(np.int64) - (1 << 32))
MASK = T - 1
HI_I32 = np.int32(np.int64(0xFFFF0000) - (1 << 32))

_b = np.exp((np.log(2048) - np.log(16)) / (L - 1))
_RES = np.floor(16 * _b ** np.arange(L)).astype(np.int64)

NC, NS, LANES = 2, 16, 16
NW = NC * NS
C = 256
NPW = N // NW
NCHUNK = NPW // C
IDXB = 4 * C
GRPB = IDXB * 8

NCACHE = 8
CELLW = [int(r) + 1 for r in _RES[:NCACHE]]
CELLS = [w * w for w in CELLW]
CELLS16 = [-(-c // 16) * 16 for c in CELLS]
GBASE = [0] * NCACHE
for _i in range(1, NCACHE):
    GBASE[_i] = GBASE[_i - 1] + CELLS16[_i - 1]
GRID_WORDS = GBASE[-1] + CELLS16[-1] + 128


def _encode_body(xT_hbm, resb_hbm, tabu_hbm, out_hbm, xv, idxv, b8v, grpv,
                 featv, resv, gridv, sem):
    wid = lax.axis_index("s") * NC + lax.axis_index("c")
    base0 = wid * NPW
    pltpu.sync_copy(resb_hbm, resv)
    iota = lax.iota(jnp.int32, LANES)

    def unpack0(u):
        return plsc.bitcast(u << 16, jnp.float32)

    def unpack1(u):
        return plsc.bitcast(u & HI_I32, jnp.float32)

    for lc in range(NCACHE):
        W = CELLW[lc]
        ncell = CELLS[lc]
        gb = GBASE[lc]
        nsg = -(-ncell // 128)

        @pl.loop(0, nsg)
        def _sg(sg, _lc=lc, _W=W, _nc=ncell, _gb=gb):
            cbase = sg * 128
            for g in range(8):
                cell = jnp.minimum(cbase + g * LANES + iota, _nc - 1)
                gy = cell // _W
                gx = cell - gy * _W
                h = ((gx ^ (gy * PRIME_I32)) & MASK) + _lc * T
                idxv[pl.ds(g * LANES, LANES)] = h

            @pl.loop(0, 8)
            def _q(q):
                hv = idxv[pl.ds(LANES * q, LANES)]
                for t in range(LANES):
                    b8 = (hv[t] // 8) * 8
                    pltpu.async_copy(
                        tabu_hbm.at[pl.ds(b8, 8)],
                        grpv.at[pl.ds((LANES * q + t) * 8, 8)], sem)

            pltpu.make_async_copy(tabu_hbm.at[pl.ds(0, 1024)],
                                  grpv.at[pl.ds(0, 1024)], sem).wait()
            for g in range(8):
                hv = idxv[pl.ds(g * LANES, LANES)]
                e = (g * LANES + iota) * 8 + (hv & 7)
                gridv[pl.ds(_gb + cbase + g * LANES, LANES)] = \
                    plsc.load_gather(grpv, [e])

    def compute_idx(l, lb):
        rv = resv[pl.ds(l * LANES, LANES)]
        off = l * T

        @pl.loop(0, C // LANES)
        def _grp(g):
            s = g * LANES
            px = xv[pl.ds(s, LANES)] * rv
            py = xv[pl.ds(C + s, LANES)] * rv
            ix = px.astype(jnp.int32)
            iy = py.astype(jnp.int32)
            m0 = iy * PRIME_I32
            m1 = m0 + PRIME_I32
            x1 = ix + 1
            b = lb * IDXB + s
            h0 = ((ix ^ m0) & MASK) + off
            h1 = ((x1 ^ m0) & MASK) + off
            h2 = ((ix ^ m1) & MASK) + off
            h3 = ((x1 ^ m1) & MASK) + off
            idxv[pl.ds(b + 0 * C, LANES)] = h0
            idxv[pl.ds(b + 1 * C, LANES)] = h1
            idxv[pl.ds(b + 2 * C, LANES)] = h2
            idxv[pl.ds(b + 3 * C, LANES)] = h3
            m8 = jnp.int32(~7)
            bb = lb * IDXB + s
            b8v[pl.ds(bb + 0 * C, LANES)] = h0 & m8
            b8v[pl.ds(bb + 1 * C, LANES)] = h1 & m8
            b8v[pl.ds(bb + 2 * C, LANES)] = h2 & m8
            b8v[pl.ds(bb + 3 * C, LANES)] = h3 & m8

    def fire(lb):
        @pl.loop(0, IDXB // LANES, unroll=4)
        def _q(q):
            bv = b8v[pl.ds(lb * IDXB + LANES * q, LANES)]
            gbase = lb * GRPB + LANES * q * 8
            for t in range(LANES):
                pltpu.async_copy(
                    tabu_hbm.at[pl.ds(pl.multiple_of(bv[t], 8), 8)],
                    grpv.at[pl.ds(gbase + t * 8, 8)], sem)

    def drain():
        pltpu.make_async_copy(tabu_hbm.at[pl.ds(0, GRPB)],
                              grpv.at[pl.ds(0, GRPB)], sem).wait()

    def interp(lp, pb):
        rv = resv[pl.ds(lp * LANES, LANES)]

        @pl.loop(0, C // LANES)
        def _grp(g):
            s = g * LANES
            px = xv[pl.ds(s, LANES)] * rv
            py = xv[pl.ds(C + s, LANES)] * rv
            fx = px - px.astype(jnp.int32).astype(jnp.float32)
            fy = py - py.astype(jnp.int32).astype(jnp.float32)
            wx0 = 1.0 - fx
            wy0 = 1.0 - fy
            ws = (wx0 * wy0, fx * wy0, wx0 * fy, fx * fy)
            acc0 = jnp.zeros((LANES,), jnp.float32)
            acc1 = jnp.zeros((LANES,), jnp.float32)
            for c in range(4):
                hv = idxv[pl.ds(pb * IDXB + c * C + s, LANES)]
                eidx = pb * GRPB + (c * C + s) * 8 + iota * 8 + (hv & 7)
                u = plsc.load_gather(grpv, [eidx])
                f0 = plsc.bitcast(u << 16, jnp.float32)
                f1 = plsc.bitcast(u & HI_I32, jnp.float32)
                acc0 = acc0 + f0 * ws[c]
                acc1 = acc1 + f1 * ws[c]
            fidx = (s + iota) * (2 * L) + 2 * lp
            plsc.store_scatter(featv, [fidx], acc0)
            plsc.store_scatter(featv, [fidx + 1], acc1)

    def interp_cached(lc):
        rv = resv[pl.ds(lc * LANES, LANES)]
        W = CELLW[lc]
        gb = GBASE[lc]

        @pl.loop(0, C // LANES)
        def _grp(g):
            s = g * LANES
            px = xv[pl.ds(s, LANES)] * rv
            py = xv[pl.ds(C + s, LANES)] * rv
            ix = px.astype(jnp.int32)
            iy = py.astype(jnp.int32)
            fx = px - ix.astype(jnp.float32)
            fy = py - iy.astype(jnp.float32)
            wx0 = 1.0 - fx
            wy0 = 1.0 - fy
            c00 = iy * W + ix + gb
            u00 = plsc.load_gather(gridv, [c00])
            u10 = plsc.load_gather(gridv, [c00 + 1])
            u01 = plsc.load_gather(gridv, [c00 + W])
            u11 = plsc.load_gather(gridv, [c00 + W + 1])
            w00 = wx0 * wy0
            w10 = fx * wy0
            w01 = wx0 * fy
            w11 = fx * fy
            acc0 = (unpack0(u00) * w00 + unpack0(u10) * w10
                    + unpack0(u01) * w01 + unpack0(u11) * w11)
            acc1 = (unpack1(u00) * w00 + unpack1(u10) * w10
                    + unpack1(u01) * w01 + unpack1(u11) * w11)
            fidx = (s + iota) * (2 * L) + 2 * lc
            plsc.store_scatter(featv, [fidx], acc0)
            plsc.store_scatter(featv, [fidx + 1], acc1)

    @pl.loop(0, NCHUNK)
    def _chunk(ci):
        base = base0 + ci * C
        pltpu.sync_copy(xT_hbm.at[0, pl.ds(base, C)], xv.at[pl.ds(0, C)])
        pltpu.sync_copy(xT_hbm.at[1, pl.ds(base, C)], xv.at[pl.ds(C, C)])

        compute_idx(NCACHE, NCACHE & 1)
        fire(NCACHE & 1)

        for lc in range(NCACHE):
            interp_cached(lc)

        @pl.loop(NCACHE + 1, L)
        def _lvl(l):
            lb = l & 1
            compute_idx(l, lb)
            fire(lb)
            drain()
            interp(l - 1, 1 - lb)

        drain()
        interp(L - 1, (L - 1) & 1)

        pltpu.sync_copy(featv, out_hbm.at[pl.ds(base * 2 * L, C * 2 * L)])


def _encode_sc(xT, tabu):
    mesh = plsc.VectorSubcoreMesh(core_axis_name="c", subcore_axis_name="s",
                                  num_cores=NC, num_subcores=NS)
    resb = jnp.asarray(
        np.broadcast_to(_RES.astype(np.float32)[:, None],
                        (L, LANES)).reshape(-1))
    enc = pl.kernel(
        _encode_body,
        out_type=jax.ShapeDtypeStruct((N * 2 * L,), jnp.float32),
        mesh=mesh,
        scratch_types=[
            pltpu.VMEM((2 * C,), jnp.float32),
            pltpu.VMEM((2 * IDXB,), jnp.int32),
            pltpu.VMEM((2 * IDXB,), jnp.int32),
            pltpu.VMEM((2 * GRPB,), jnp.int32),
            pltpu.VMEM((C * 2 * L,), jnp.float32),
            pltpu.VMEM((L * LANES,), jnp.float32),
            pltpu.VMEM((GRID_WORDS,), jnp.int32),
            pltpu.SemaphoreType.DMA,
        ],
        compiler_params=pltpu.CompilerParams(needs_layout_passes=False),
    )
    return enc(xT, resb, tabu).reshape(N, 2 * L)


def _mlp_body(f_ref, w1_ref, b1_ref, w2_ref, b2_ref, w3_ref, b3_ref, o_ref):
    h = f_ref[...]
    h = jnp.maximum(
        jnp.dot(h, w1_ref[...], preferred_element_type=jnp.float32)
        + b1_ref[...], 0.0)
    h = jnp.maximum(
        jnp.dot(h, w2_ref[...], preferred_element_type=jnp.float32)
        + b2_ref[...], 0.0)
    y = jnp.dot(h, w3_ref[...], preferred_element_type=jnp.float32) \
        + b3_ref[...]
    o_ref[...] = jax.nn.sigmoid(y)


def _mlp_tc(feats, Ws, bs):
    NB = 4096
    grid = (N // NB,)
    full = lambda shape: pl.BlockSpec(shape, lambda i: (0, 0))
    return pl.pallas_call(
        _mlp_body,
        grid=grid,
        in_specs=[
            pl.BlockSpec((NB, 2 * L), lambda i: (i, 0)),
            full((2 * L, WIDTH)),
            full((1, WIDTH)),
            full((WIDTH, WIDTH)),
            full((1, WIDTH)),
            full((WIDTH, OUT)),
            full((1, OUT)),
        ],
        out_specs=pl.BlockSpec((NB, OUT), lambda i: (i, 0)),
        out_shape=jax.ShapeDtypeStruct((N, OUT), jnp.float32),
    )(feats, Ws[0], bs[0].reshape(1, -1), Ws[1], bs[1].reshape(1, -1),
      Ws[2], bs[2].reshape(1, -1))


def kernel(x, table, Ws, bs):
    xT = x.T
    tb = table.reshape(L * T, F).astype(jnp.bfloat16)
    bits = lax.bitcast_convert_type(tb, jnp.uint16)
    u = bits[:, 0].astype(jnp.uint32) | (bits[:, 1].astype(jnp.uint32) << 16)
    tabu = lax.bitcast_convert_type(u, jnp.int32)
    feats = _encode_sc(xT, tabu)
    return _mlp_tc(feats, Ws, bs)
